# baseline (speedup 1.0000x reference)
import jax
import jax.numpy as jnp
from jax import lax
from jax.experimental import pallas as pl
from jax.experimental.pallas import tpu as pltpu
from jax.experimental.pallas import tpu_sc as plsc

_VOCAB = 100000
_DIM = 64
_B = 4096
_L = 50
_NF = 4
_NC, _NS, _LANES = 2, 16, 16
_NW = _NC * _NS
_ROWS_PER_W = _B // _NW
_G = 16
_CHUNKS = _ROWS_PER_W // _G
_IDS = _G * _L
_NSEG = _IDS // 128
_REM = _IDS - _NSEG * 128
_FSTRIDE = _B * _L


def _pool_body(ids_hbm, tbl, out_hbm, ibuf0, ibuf1, gbuf0, gbuf1, out_v,
               gsem0, gsem1):
    ibufs = (ibuf0, ibuf1)
    gbufs = (gbuf0, gbuf1)
    gsems = (gsem0, gsem1)
    wid = lax.axis_index("s") * _NC + lax.axis_index("c")
    iota = lax.broadcasted_iota(jnp.int32, (_LANES,), 0)
    tail_mask = iota < (_L - 3 * _LANES)

    def issue(f, base, slot):
        pltpu.sync_copy(
            ids_hbm.at[pl.ds(f * _FSTRIDE + base * _L, _IDS)], ibufs[slot])
        for p in range(_NSEG):
            pltpu.async_copy(
                tbl.at[ibufs[slot].at[pl.ds(p * 128, 128)]],
                gbufs[slot].at[pl.ds(p * 128, 128)], gsems[slot])
        pltpu.async_copy(
            tbl.at[ibufs[slot].at[pl.ds(_NSEG * 128, _REM)]],
            gbufs[slot].at[pl.ds(_NSEG * 128, _REM)], gsems[slot])

    def drain_gather(slot):
        for p in range(_NSEG):
            pltpu.make_async_copy(
                tbl.at[ibufs[slot].at[pl.ds(p * 128, 128)]],
                gbufs[slot].at[pl.ds(p * 128, 128)], gsems[slot]).wait()
        pltpu.make_async_copy(
            tbl.at[ibufs[slot].at[pl.ds(_NSEG * 128, _REM)]],
            gbufs[slot].at[pl.ds(_NSEG * 128, _REM)], gsems[slot]).wait()

    issue(0, wid * _ROWS_PER_W, 0)

    @pl.loop(0, _CHUNKS)
    def _chunk(ch):
        base = pl.multiple_of(wid * _ROWS_PER_W + ch * _G, _G)

        for f in range(_NF):
            slot = f % 2
            drain_gather(slot)

            if f < _NF - 1:
                issue(f + 1, base, 1 - slot)
            else:
                @pl.when(ch < _CHUNKS - 1)
                def _():
                    issue(0, base + _G, 1 - slot)

            pad = jnp.int32(f * _VOCAB)

            @pl.loop(0, _G)
            def _row(g):
                rbase = g * _L
                cnt = jnp.zeros((_LANES,), jnp.int32)
                for k in range(3):
                    x = plsc.load_gather(
                        ibufs[slot], [rbase + k * _LANES + iota])
                    cnt = cnt + plsc.all_reduce_population_count(x != pad)
                x = plsc.load_gather(
                    ibufs[slot], [rbase + 3 * _LANES + iota], mask=tail_mask)
                cnt = cnt + plsc.all_reduce_population_count(
                    (x != pad) & tail_mask)
                scale = 1.0 / jnp.maximum(cnt.astype(jnp.float32), 1.0)

                def _sum(j, acc):
                    r = rbase + j
                    return tuple(
                        acc[c] + gbufs[slot][r, pl.ds(c * _LANES, _LANES)]
                        for c in range(4))

                acc = lax.fori_loop(
                    0, _L, _sum,
                    tuple(jnp.zeros((_LANES,), jnp.float32)
                          for _ in range(4)),
                    unroll=10)
                for c in range(4):
                    out_v[g, pl.ds(f * _DIM + c * _LANES, _LANES)] = (
                        acc[c] * scale)

        pltpu.sync_copy(out_v, out_hbm.at[pl.ds(base, _G)])


@jax.jit
def kernel(item_ids, cat_ids, brand_ids, shop_ids, T_item, T_cat, T_brand,
           T_shop):
    mesh = plsc.VectorSubcoreMesh(core_axis_name="c", subcore_axis_name="s")
    run = pl.kernel(
        _pool_body,
        out_type=jax.ShapeDtypeStruct((_B, _NF * _DIM), jnp.float32),
        mesh=mesh,
        compiler_params=pltpu.CompilerParams(
            needs_layout_passes=False, use_tc_tiling_on_sc=False),
        scratch_types=[
            pltpu.VMEM((_IDS,), jnp.int32),
            pltpu.VMEM((_IDS,), jnp.int32),
            pltpu.VMEM((_IDS, _DIM), jnp.float32),
            pltpu.VMEM((_IDS, _DIM), jnp.float32),
            pltpu.VMEM((_G, _NF * _DIM), jnp.float32),
            pltpu.SemaphoreType.DMA,
            pltpu.SemaphoreType.DMA,
        ],
    )
    offs = (jnp.arange(_NF, dtype=jnp.int32) * _VOCAB)[:, None, None]
    ids_all = (jnp.stack([item_ids, cat_ids, brand_ids, shop_ids]) + offs
               ).reshape(-1)
    tbl_all = jnp.concatenate([T_item, T_cat, T_brand, T_shop], axis=0)
    return run(ids_all, tbl_all)


# speedup vs baseline: 1.5754x; 1.5754x over previous
import jax
import jax.numpy as jnp
from jax import lax
from jax.experimental import pallas as pl
from jax.experimental.pallas import tpu as pltpu
from jax.experimental.pallas import tpu_sc as plsc

_VOCAB = 100000
_DIM = 64
_B = 4096
_L = 50
_NF = 4
_NC, _NS, _LANES = 2, 16, 16
_NW = _NC * _NS
_ROWS_PER_W = _B // _NW
_G = 16
_CHUNKS = _ROWS_PER_W // _G
_IDS = _G * _L


def _pool_body(i0, i1, i2, i3, t0, t1, t2, t3, out_hbm,
               ibuf0, ibuf1, gbuf0, gbuf1, out_v, gsem0, gsem1):
    ids_hbm = (i0, i1, i2, i3)
    tbls = (t0, t1, t2, t3)
    ibufs = (ibuf0, ibuf1)
    gbufs = (gbuf0, gbuf1)
    gsems = (gsem0, gsem1)
    wid = lax.axis_index("s") * _NC + lax.axis_index("c")
    iota = lax.broadcasted_iota(jnp.int32, (_LANES,), 0)
    tail_mask = iota < (_L - 3 * _LANES)

    def issue(f, base, slot):
        pltpu.sync_copy(ids_hbm[f].at[pl.ds(base, _G), :], ibufs[slot])
        for g in range(_G):
            pltpu.async_copy(
                tbls[f].at[ibufs[slot].at[g]],
                gbufs[slot].at[pl.ds(g * _L, _L)], gsems[slot])

    def drain_gather(f, slot):
        for g in range(_G):
            pltpu.make_async_copy(
                tbls[f].at[ibufs[slot].at[g]],
                gbufs[slot].at[pl.ds(g * _L, _L)], gsems[slot]).wait()

    issue(0, wid * _ROWS_PER_W, 0)

    @pl.loop(0, _CHUNKS)
    def _chunk(ch):
        base = pl.multiple_of(wid * _ROWS_PER_W + ch * _G, _G)

        for f in range(_NF):
            slot = f % 2
            drain_gather(f, slot)

            if f < _NF - 1:
                issue(f + 1, base, 1 - slot)
            else:
                @pl.when(ch < _CHUNKS - 1)
                def _():
                    issue(0, base + _G, 1 - slot)

            @pl.loop(0, _G)
            def _row(g):
                cnt = jnp.zeros((_LANES,), jnp.int32)
                for k in range(3):
                    x = plsc.load_gather(
                        ibufs[slot], [g + jnp.zeros((_LANES,), jnp.int32),
                                      k * _LANES + iota])
                    cnt = cnt + plsc.all_reduce_population_count(x != 0)
                x = plsc.load_gather(
                    ibufs[slot], [g + jnp.zeros((_LANES,), jnp.int32),
                                  3 * _LANES + iota], mask=tail_mask)
                cnt = cnt + plsc.all_reduce_population_count(
                    (x != 0) & tail_mask)
                scale = 1.0 / jnp.maximum(cnt.astype(jnp.float32), 1.0)

                rbase = g * _L

                def _sum(j, acc):
                    r = rbase + j
                    return tuple(
                        acc[c] + gbufs[slot][r, pl.ds(c * _LANES, _LANES)]
                        for c in range(4))

                acc = lax.fori_loop(
                    0, _L, _sum,
                    tuple(jnp.zeros((_LANES,), jnp.float32)
                          for _ in range(4)),
                    unroll=10)
                for c in range(4):
                    out_v[g, pl.ds(f * _DIM + c * _LANES, _LANES)] = (
                        acc[c] * scale)

        pltpu.sync_copy(out_v, out_hbm.at[pl.ds(base, _G)])


@jax.jit
def kernel(item_ids, cat_ids, brand_ids, shop_ids, T_item, T_cat, T_brand,
           T_shop):
    mesh = plsc.VectorSubcoreMesh(core_axis_name="c", subcore_axis_name="s")
    run = pl.kernel(
        _pool_body,
        out_type=jax.ShapeDtypeStruct((_B, _NF * _DIM), jnp.float32),
        mesh=mesh,
        compiler_params=pltpu.CompilerParams(
            needs_layout_passes=False, use_tc_tiling_on_sc=False),
        scratch_types=[
            pltpu.VMEM((_G, _L), jnp.int32),
            pltpu.VMEM((_G, _L), jnp.int32),
            pltpu.VMEM((_IDS, _DIM), jnp.float32),
            pltpu.VMEM((_IDS, _DIM), jnp.float32),
            pltpu.VMEM((_G, _NF * _DIM), jnp.float32),
            pltpu.SemaphoreType.DMA,
            pltpu.SemaphoreType.DMA,
        ],
    )
    return run(item_ids, cat_ids, brand_ids, shop_ids,
               T_item, T_cat, T_brand, T_shop)


# speedup vs baseline: 1.7414x; 1.1053x over previous
import jax
import jax.numpy as jnp
from jax import lax
from jax.experimental import pallas as pl
from jax.experimental.pallas import tpu as pltpu
from jax.experimental.pallas import tpu_sc as plsc

_VOCAB = 100000
_DIM = 64
_B = 4096
_L = 50
_NF = 4
_NC, _NS, _LANES = 2, 16, 16
_NW = _NC * _NS
_ROWS_PER_W = _B // _NW
_G = 16
_CHUNKS = _ROWS_PER_W // _G
_IDS = _G * _L


def _pool_body(ids_hbm, tbl, out_hbm, ibuf0, ibuf1, gbuf0, gbuf1, out_v,
               gsem0, gsem1):
    ibufs = (ibuf0, ibuf1)
    gbufs = (gbuf0, gbuf1)
    gsems = (gsem0, gsem1)
    wid = lax.axis_index("s") * _NC + lax.axis_index("c")
    iota = lax.broadcasted_iota(jnp.int32, (_LANES,), 0)
    tail_mask = iota < (_L - 3 * _LANES)

    def issue(base, slot):
        pltpu.sync_copy(ids_hbm.at[pl.ds(base, _G), :], ibufs[slot])
        for g in range(_G):
            pltpu.async_copy(
                tbl.at[ibufs[slot].at[g]],
                gbufs[slot].at[pl.ds(g * _L, _L)], gsems[slot])

    def drain_gather(slot):
        for g in range(_G):
            pltpu.make_async_copy(
                tbl.at[ibufs[slot].at[g]],
                gbufs[slot].at[pl.ds(g * _L, _L)], gsems[slot]).wait()

    issue(wid * _ROWS_PER_W, 0)

    for ch in range(_CHUNKS):
        base = pl.multiple_of(wid * _ROWS_PER_W + ch * _G, _G)
        slot = ch % 2
        drain_gather(slot)

        if ch < _CHUNKS - 1:
            issue(base + _G, 1 - slot)

        @pl.loop(0, _G)
        def _row(g):
            cnt = jnp.zeros((_LANES,), jnp.int32)
            for k in range(3):
                x = plsc.load_gather(
                    ibufs[slot], [g + jnp.zeros((_LANES,), jnp.int32),
                                  k * _LANES + iota])
                cnt = cnt + plsc.all_reduce_population_count(x != 0)
            x = plsc.load_gather(
                ibufs[slot], [g + jnp.zeros((_LANES,), jnp.int32),
                              3 * _LANES + iota], mask=tail_mask)
            cnt = cnt + plsc.all_reduce_population_count(
                (x != 0) & tail_mask)
            scale = 1.0 / jnp.maximum(cnt.astype(jnp.float32), 1.0)

            rbase = g * _L

            def _sum(j, acc):
                r = rbase + j
                return tuple(
                    acc[c] + gbufs[slot][r, pl.ds(c * _LANES, _LANES)]
                    for c in range(4))

            acc = lax.fori_loop(
                0, _L, _sum,
                tuple(jnp.zeros((_LANES,), jnp.float32) for _ in range(4)),
                unroll=10)
            for c in range(4):
                out_v[g, pl.ds(c * _LANES, _LANES)] = acc[c] * scale

        pltpu.sync_copy(out_v, out_hbm.at[pl.ds(base, _G)])


@jax.jit
def kernel(item_ids, cat_ids, brand_ids, shop_ids, T_item, T_cat, T_brand,
           T_shop):
    mesh = plsc.VectorSubcoreMesh(core_axis_name="c", subcore_axis_name="s")
    run = pl.kernel(
        _pool_body,
        out_type=jax.ShapeDtypeStruct((_B, _DIM), jnp.float32),
        mesh=mesh,
        compiler_params=pltpu.CompilerParams(
            needs_layout_passes=False, use_tc_tiling_on_sc=False),
        scratch_types=[
            pltpu.VMEM((_G, _L), jnp.int32),
            pltpu.VMEM((_G, _L), jnp.int32),
            pltpu.VMEM((_IDS, _DIM), jnp.float32),
            pltpu.VMEM((_IDS, _DIM), jnp.float32),
            pltpu.VMEM((_G, _DIM), jnp.float32),
            pltpu.SemaphoreType.DMA,
            pltpu.SemaphoreType.DMA,
        ],
    )
    pooled = [run(ids, tbl) for ids, tbl in
              ((item_ids, T_item), (cat_ids, T_cat),
               (brand_ids, T_brand), (shop_ids, T_shop))]
    return jnp.concatenate(pooled, axis=1)


# speedup vs baseline: 1.7550x; 1.0078x over previous
import jax
import jax.numpy as jnp
from jax import lax
from jax.experimental import pallas as pl
from jax.experimental.pallas import tpu as pltpu
from jax.experimental.pallas import tpu_sc as plsc

_VOCAB = 100000
_DIM = 64
_B = 4096
_L = 50
_NF = 4
_NC, _NS, _LANES = 2, 16, 16
_NW = _NC * _NS
_ROWS_PER_W = _B // _NW
_G = 16
_CHUNKS = _ROWS_PER_W // _G
_IDS = _G * _L
_NSEG = _IDS // 128
_REM = _IDS - _NSEG * 128


def _pool_body(ids_hbm, tbl, out_hbm, ibuf0, ibuf1, gbuf0, gbuf1, out_v,
               gsem0, gsem1):
    ibufs = (ibuf0, ibuf1)
    gbufs = (gbuf0, gbuf1)
    gsems = (gsem0, gsem1)
    wid = lax.axis_index("s") * _NC + lax.axis_index("c")
    iota = lax.broadcasted_iota(jnp.int32, (_LANES,), 0)
    tail_mask = iota < (_L - 3 * _LANES)

    def issue(base, slot):
        pltpu.sync_copy(ids_hbm.at[pl.ds(base * _L, _IDS)], ibufs[slot])
        for p in range(_NSEG):
            pltpu.async_copy(
                tbl.at[ibufs[slot].at[pl.ds(p * 128, 128)]],
                gbufs[slot].at[pl.ds(p * 128, 128)], gsems[slot])
        pltpu.async_copy(
            tbl.at[ibufs[slot].at[pl.ds(_NSEG * 128, _REM)]],
            gbufs[slot].at[pl.ds(_NSEG * 128, _REM)], gsems[slot])

    def drain_gather(slot):
        for p in range(_NSEG):
            pltpu.make_async_copy(
                tbl.at[ibufs[slot].at[pl.ds(p * 128, 128)]],
                gbufs[slot].at[pl.ds(p * 128, 128)], gsems[slot]).wait()
        pltpu.make_async_copy(
            tbl.at[ibufs[slot].at[pl.ds(_NSEG * 128, _REM)]],
            gbufs[slot].at[pl.ds(_NSEG * 128, _REM)], gsems[slot]).wait()

    issue(wid * _ROWS_PER_W, 0)

    for ch in range(_CHUNKS):
        base = pl.multiple_of(wid * _ROWS_PER_W + ch * _G, _G)
        slot = ch % 2
        drain_gather(slot)

        if ch < _CHUNKS - 1:
            issue(base + _G, 1 - slot)

        @pl.loop(0, _G)
        def _row(g):
            rbase = g * _L
            cnt = jnp.zeros((_LANES,), jnp.int32)
            for k in range(3):
                x = plsc.load_gather(ibufs[slot], [rbase + k * _LANES + iota])
                cnt = cnt + plsc.all_reduce_population_count(x != 0)
            x = plsc.load_gather(
                ibufs[slot], [rbase + 3 * _LANES + iota], mask=tail_mask)
            cnt = cnt + plsc.all_reduce_population_count(
                (x != 0) & tail_mask)
            scale = 1.0 / jnp.maximum(cnt.astype(jnp.float32), 1.0)

            def _sum(j, acc):
                r = rbase + j
                return tuple(
                    acc[c] + gbufs[slot][r, pl.ds(c * _LANES, _LANES)]
                    for c in range(4))

            acc = lax.fori_loop(
                0, _L, _sum,
                tuple(jnp.zeros((_LANES,), jnp.float32) for _ in range(4)),
                unroll=10)
            for c in range(4):
                out_v[g, pl.ds(c * _LANES, _LANES)] = acc[c] * scale

        pltpu.sync_copy(out_v, out_hbm.at[pl.ds(base, _G)])


@jax.jit
def kernel(item_ids, cat_ids, brand_ids, shop_ids, T_item, T_cat, T_brand,
           T_shop):
    mesh = plsc.VectorSubcoreMesh(core_axis_name="c", subcore_axis_name="s")
    run = pl.kernel(
        _pool_body,
        out_type=jax.ShapeDtypeStruct((_B, _DIM), jnp.float32),
        mesh=mesh,
        compiler_params=pltpu.CompilerParams(
            needs_layout_passes=False, use_tc_tiling_on_sc=False),
        scratch_types=[
            pltpu.VMEM((_IDS,), jnp.int32),
            pltpu.VMEM((_IDS,), jnp.int32),
            pltpu.VMEM((_IDS, _DIM), jnp.float32),
            pltpu.VMEM((_IDS, _DIM), jnp.float32),
            pltpu.VMEM((_G, _DIM), jnp.float32),
            pltpu.SemaphoreType.DMA,
            pltpu.SemaphoreType.DMA,
        ],
    )
    pooled = [run(ids.reshape(-1), tbl) for ids, tbl in
              ((item_ids, T_item), (cat_ids, T_cat),
               (brand_ids, T_brand), (shop_ids, T_shop))]
    return jnp.concatenate(pooled, axis=1)
